# baseline (speedup 1.0000x reference)
import jax
import jax.numpy as jnp
from jax.experimental import pallas as pl


def _dice(x, alpha):
    p = jax.nn.sigmoid(x)
    return p * x + (1.0 - p) * alpha * x


def _gru_scan(xs, mask, Wx, Wh, b):
    h0 = jnp.zeros((xs.shape[1], xs.shape[2]), jnp.float32)
    def step(h, inp):
        x, m = inp
        gx = x @ Wx + b
        gh = h @ Wh
        xr, xz, xn = jnp.split(gx, 3, axis=-1)
        hr, hz, hn = jnp.split(gh, 3, axis=-1)
        r = jax.nn.sigmoid(xr + hr)
        z = jax.nn.sigmoid(xz + hz)
        n = jnp.tanh(xn + r * hn)
        hnew = (1.0 - z) * n + z * h
        hnew = jnp.where(m, hnew, h)
        return hnew, hnew
    _, hs = jax.lax.scan(step, h0, (xs, mask))
    return hs


def _augru_scan(xs, att, mask, Wx, Wh, b):
    h0 = jnp.zeros((xs.shape[1], xs.shape[2]), jnp.float32)
    def step(h, inp):
        x, a, m = inp
        gx = x @ Wx + b
        gh = h @ Wh
        xr, xz, xn = jnp.split(gx, 3, axis=-1)
        hr, hz, hn = jnp.split(gh, 3, axis=-1)
        r = jax.nn.sigmoid(xr + hr)
        u = jax.nn.sigmoid(xz + hz)
        n = jnp.tanh(xn + r * hn)
        u = a * u
        hnew = (1.0 - u) * h + u * n
        hnew = jnp.where(m, hnew, h)
        return hnew, None
    hT, _ = jax.lax.scan(step, h0, (xs, att, mask))
    return hT


def _passthrough(x_ref, o_ref):
    o_ref[...] = x_ref[...]


def kernel(pos_item_seq, neg_item_seq, target_item, non_seq_ids, seq_len, params):
    emb = params['emb_table']
    Bn, Ln = pos_item_seq.shape
    pos_e = jnp.take(emb, pos_item_seq, axis=0)
    neg_e = jnp.take(emb, neg_item_seq, axis=0)
    tgt_e = jnp.take(emb, target_item, axis=0)
    ns_e = jnp.take(emb, non_seq_ids, axis=0).reshape(Bn, -1)
    mask_bl = jnp.arange(Ln)[None, :] < seq_len[:, None]
    xs = jnp.transpose(pos_e, (1, 0, 2))
    m_lb = jnp.transpose(mask_bl, (1, 0))[:, :, None]
    hs = _gru_scan(xs, m_lb, params['gru_Wx'], params['gru_Wh'], params['gru_b'])
    interests = jnp.transpose(hs, (1, 0, 2))
    def aux_net(a, b_):
        x = jnp.concatenate([a, b_], axis=-1)
        x = jax.nn.sigmoid(x @ params['aux_W1'] + params['aux_b1'])
        x = jax.nn.sigmoid(x @ params['aux_W2'] + params['aux_b2'])
        x = x @ params['aux_W3'] + params['aux_b3']
        return jax.nn.sigmoid(x)[..., 0]
    eps = 1e-7
    click_p = aux_net(interests[:, :-1, :], pos_e[:, 1:, :])
    noclick_p = aux_net(interests[:, :-1, :], neg_e[:, 1:, :])
    aux_mask = (jnp.arange(Ln - 1)[None, :] < (seq_len[:, None] - 1)).astype(jnp.float32)
    aux_loss = jnp.sum(-(jnp.log(click_p + eps) + jnp.log(1.0 - noclick_p + eps)) * aux_mask) / jnp.maximum(jnp.sum(aux_mask), 1.0)
    q = jnp.broadcast_to(tgt_e[:, None, :], interests.shape)
    att_in = jnp.concatenate([q, interests, q - interests, q * interests], axis=-1)
    ah = jax.nn.sigmoid(att_in @ params['att_W1'] + params['att_b1'])
    scores = (ah @ params['att_W2'] + params['att_b2'])[..., 0]
    scores = jnp.where(mask_bl, scores, -1e9)
    att = jax.nn.softmax(scores, axis=-1)
    att_lb = jnp.transpose(att, (1, 0))[:, :, None]
    evo = _augru_scan(hs, att_lb, m_lb, params['augru_Wx'], params['augru_Wh'], params['augru_b'])
    din = jnp.concatenate([tgt_e, ns_e, evo], axis=-1)
    x = _dice(din @ params['dnn_W1'] + params['dnn_b1'], params['dnn_a1'])
    x = _dice(x @ params['dnn_W2'] + params['dnn_b2'], params['dnn_a2'])
    x = _dice(x @ params['dnn_W3'] + params['dnn_b3'], params['dnn_a3'])
    preds = jax.nn.sigmoid(x)
    preds = pl.pallas_call(
        _passthrough,
        out_shape=jax.ShapeDtypeStruct(preds.shape, preds.dtype),
    )(preds)
    return (preds, aux_loss)


# speedup vs baseline: 6.6281x; 6.6281x over previous
import jax
import jax.numpy as jnp
from jax import lax
from jax.experimental import pallas as pl
from jax.experimental.pallas import tpu as pltpu
from jax.experimental.pallas import tpu_sc as plsc

_B, _L, _NNS = 1024, 200, 7
_E = 30


_NW = 32
_TOTAL_ROWS = 2 * _B * _L + _B + _B * _NNS
_IDXW = 96
_GRP = 8
_NGRP = 17
_ROWS_PER_W = _IDXW * _GRP * _NGRP
assert _ROWS_PER_W * _NW == _TOTAL_ROWS


def _sc_gather_kernel(table_hbm, idx_hbm, out_hbm, idx_v, buf_a, buf_b, sem_a, sem_b):
    wid = lax.axis_index("s") * 2 + lax.axis_index("c")
    base = wid * _ROWS_PER_W
    pltpu.sync_copy(idx_hbm.at[wid], idx_v)

    bufs = (buf_a, buf_b)
    sems = (sem_a, sem_b)

    def fire(g):
        cur, sem = bufs[g % 2], sems[g % 2]
        for j in range(_GRP):
            pltpu.async_copy(
                table_hbm.at[idx_v.at[g * _GRP + j]],
                cur.at[pl.ds(j * _IDXW, _IDXW)], sem)

    def drain(g):
        cur, sem = bufs[g % 2], sems[g % 2]
        dst = out_hbm.at[pl.ds(base + g * (_GRP * _IDXW), _GRP * _IDXW)]
        pltpu.make_async_copy(dst, cur, sem).wait()
        pltpu.sync_copy(cur, dst)

    fire(0)
    for g in range(1, _NGRP):
        fire(g)
        drain(g - 1)
    drain(_NGRP - 1)


def _sc_gather(table, idx3):
    mesh = plsc.VectorSubcoreMesh(core_axis_name="c", subcore_axis_name="s")
    return pl.kernel(
        _sc_gather_kernel,
        mesh=mesh,
        compiler_params=pltpu.CompilerParams(use_tc_tiling_on_sc=False),
        out_type=jax.ShapeDtypeStruct((_TOTAL_ROWS, _E), jnp.float32),
        scratch_types=[
            pltpu.VMEM((_NGRP * _GRP, _IDXW), jnp.int32),
            pltpu.VMEM((_GRP * _IDXW, _E), jnp.float32),
            pltpu.VMEM((_GRP * _IDXW, _E), jnp.float32),
            pltpu.SemaphoreType.DMA,
            pltpu.SemaphoreType.DMA,
        ],
    )(table, idx3)


_BB = 256
_NBLK = _B // _BB


def _sig(x):
    return jax.nn.sigmoid(x)


def _dice(x, alpha):
    p = _sig(x)
    return p * x + (1.0 - p) * alpha * x


def _mm(a, b):
    return lax.dot_general(a, b, (((1,), (0,)), ((), ())),
                           preferred_element_type=jnp.float32)


def _tc_kernel(pos_ref, neg_ref, tgt_ref, ns_ref, sl_ref, w_refs,
               preds_ref, aux_ref, hs_ref, att_ref, acc_ref):
    i = pl.program_id(0)
    f32 = jnp.float32
    w = {k: r[...] for k, r in w_refs.items()}

    sl = sl_ref[...]
    tgt = tgt_ref[...]

    def gru_step(t, h):
        x = pos_ref[t]
        r = _sig(_mm(w['gru_Wx_r'], x) + w['gru_b_r'] + _mm(w['gru_Wh_r'], h))
        z = _sig(_mm(w['gru_Wx_z'], x) + w['gru_b_z'] + _mm(w['gru_Wh_z'], h))
        n = jnp.tanh(_mm(w['gru_Wx_n'], x) + w['gru_b_n']
                     + r * _mm(w['gru_Wh_n'], h))
        hnew = (1.0 - z) * n + z * h
        hnew = jnp.where(t < sl, hnew, h)
        hs_ref[t] = hnew
        return hnew

    h0 = jnp.zeros((_E, _BB), f32)
    lax.fori_loop(0, _L, gru_step, h0, unroll=False)

    eps = 1e-7

    def aux_step(t, acc):
        h = hs_ref[t]
        hW = _mm(w['aux_W1_h'], h)

        def head(e):
            x = _sig(hW + _mm(w['aux_W1_e'], e) + w['aux_b1'])
            x = _sig(_mm(w['aux_W2'], x) + w['aux_b2'])
            return _sig(_mm(w['aux_W3'], x) + w['aux_b3'])

        click = head(pos_ref[t + 1])
        noclick = head(neg_ref[t + 1])
        amf = (t < (sl - 1)).astype(f32)
        num = jnp.sum(-(jnp.log(click + eps) + jnp.log(1.0 - noclick + eps)) * amf)
        return (acc[0] + num, acc[1] + jnp.sum(amf))

    aux_num, aux_den = lax.fori_loop(
        0, _L - 1, aux_step, (jnp.zeros((), f32), jnp.zeros((), f32)),
        unroll=False)

    qW = _mm(w['att_W1_qd'], tgt) + w['att_b1']

    def att_step(t, _):
        h = hs_ref[t]
        ah = _sig(_mm(w['att_W1_id'], h) + _mm(w['att_W1_p'], h * tgt) + qW)
        sc = _mm(w['att_W2'], ah) + w['att_b2']
        att_ref[pl.ds(t, 1), :] = jnp.where(t < sl, sc, -1e9)
        return 0

    lax.fori_loop(0, _L, att_step, 0, unroll=False)

    scores = att_ref[...]
    sexp = jnp.exp(scores - jnp.max(scores, axis=0, keepdims=True))
    att_ref[...] = sexp / jnp.sum(sexp, axis=0, keepdims=True)

    def augru_step(t, h):
        x = hs_ref[t]
        a = att_ref[pl.ds(t, 1), :]
        r = _sig(_mm(w['augru_Wx_r'], x) + w['augru_b_r']
                 + _mm(w['augru_Wh_r'], h))
        u = _sig(_mm(w['augru_Wx_z'], x) + w['augru_b_z']
                 + _mm(w['augru_Wh_z'], h)) * a
        n = jnp.tanh(_mm(w['augru_Wx_n'], x) + w['augru_b_n']
                     + r * _mm(w['augru_Wh_n'], h))
        hnew = (1.0 - u) * h + u * n
        return jnp.where(t < sl, hnew, h)

    evo = lax.fori_loop(0, _L, augru_step, h0, unroll=False)

    x = (_mm(w['dnn_W1_t'], tgt) + _mm(w['dnn_W1_n'], ns_ref[...])
         + _mm(w['dnn_W1_e'], evo) + w['dnn_b1'])
    x = _dice(x, w['dnn_a1'])
    x = _dice(_mm(w['dnn_W2'], x) + w['dnn_b2'], w['dnn_a2'])
    x = _dice(_mm(w['dnn_W3'], x) + w['dnn_b3'], w['dnn_a3'])
    preds_ref[...] = _sig(x)

    @pl.when(i == 0)
    def _():
        acc_ref[0] = aux_num
        acc_ref[1] = aux_den

    @pl.when(i > 0)
    def _():
        acc_ref[0] += aux_num
        acc_ref[1] += aux_den

    @pl.when(i == _NBLK - 1)
    def _():
        aux_ref[...] = (acc_ref[0] / jnp.maximum(acc_ref[1], 1.0)).reshape(1, 1)


def _tc_network(pos_tm, neg_tm, tgt_e, ns_e, seq_len, params):
    f32 = jnp.float32
    E = _E

    weights = {}
    for pre in ('gru', 'augru'):
        wx = params[pre + '_Wx']
        wh = params[pre + '_Wh']
        b = params[pre + '_b']
        for k, name in enumerate(('r', 'z', 'n')):
            weights[pre + '_Wx_' + name] = wx[:, k * E:(k + 1) * E].T
            weights[pre + '_Wh_' + name] = wh[:, k * E:(k + 1) * E].T
            weights[pre + '_b_' + name] = b[k * E:(k + 1) * E].reshape(E, 1)
    aw1 = params['aux_W1']
    weights['aux_W1_h'] = aw1[:E].T
    weights['aux_W1_e'] = aw1[E:].T
    weights['aux_W2'] = params['aux_W2'].T
    weights['aux_W3'] = params['aux_W3'].T
    tw1 = params['att_W1']
    weights['att_W1_qd'] = (tw1[:E] + tw1[2 * E:3 * E]).T
    weights['att_W1_id'] = (tw1[E:2 * E] - tw1[2 * E:3 * E]).T
    weights['att_W1_p'] = tw1[3 * E:].T
    weights['att_W2'] = params['att_W2'].T
    dw1 = params['dnn_W1']
    weights['dnn_W1_t'] = dw1[:E].T
    weights['dnn_W1_n'] = dw1[E:E + _NNS * E].T
    weights['dnn_W1_e'] = dw1[E + _NNS * E:].T
    weights['dnn_W2'] = params['dnn_W2'].T
    weights['dnn_W3'] = params['dnn_W3'].T
    for nm in ('aux_b1', 'aux_b2', 'aux_b3', 'att_b1', 'att_b2',
               'dnn_b1', 'dnn_a1', 'dnn_b2', 'dnn_a2', 'dnn_b3', 'dnn_a3'):
        weights[nm] = params[nm].reshape(-1, 1)

    def wspec(shape):
        return pl.BlockSpec(shape, lambda i: (0, 0))

    w_specs = {k: wspec(v.shape) for k, v in weights.items()}
    sl2 = seq_len.reshape(1, _B).astype(jnp.int32)

    preds, aux = pl.pallas_call(
        _tc_kernel,
        grid=(_NBLK,),
        in_specs=[
            pl.BlockSpec((_L, E, _BB), lambda i: (0, 0, i)),
            pl.BlockSpec((_L, E, _BB), lambda i: (0, 0, i)),
            pl.BlockSpec((E, _BB), lambda i: (0, i)),
            pl.BlockSpec((_NNS * E, _BB), lambda i: (0, i)),
            pl.BlockSpec((1, _BB), lambda i: (0, i)),
            w_specs,
        ],
        out_specs=[
            pl.BlockSpec((1, _BB), lambda i: (0, i)),
            pl.BlockSpec((1, 1), lambda i: (0, 0)),
        ],
        out_shape=[
            jax.ShapeDtypeStruct((1, _B), f32),
            jax.ShapeDtypeStruct((1, 1), f32),
        ],
        scratch_shapes=[
            pltpu.VMEM((_L, _E, _BB), f32),
            pltpu.VMEM((_L, _BB), f32),
            pltpu.SMEM((2,), f32),
        ],
        compiler_params=pltpu.CompilerParams(
            vmem_limit_bytes=63 * 1024 * 1024),
    )(pos_tm, neg_tm, tgt_e, ns_e, sl2, weights)
    return preds.reshape(_B, 1), aux[0, 0]


def kernel(pos_item_seq, neg_item_seq, target_item, non_seq_ids, seq_len, params):
    flat_idx = jnp.concatenate([
        pos_item_seq.T.reshape(-1),
        neg_item_seq.T.reshape(-1),
        target_item,
        non_seq_ids.reshape(-1),
    ]).astype(jnp.int32)
    idx3 = flat_idx.reshape(_NW, _NGRP * _GRP, _IDXW)
    del idx3

    rows = jnp.take(params['emb_table'], flat_idx, axis=0)

    n_seq = _B * _L
    pos_tm = rows[:n_seq].reshape(_L, _B, _E).transpose(0, 2, 1)
    neg_tm = rows[n_seq:2 * n_seq].reshape(_L, _B, _E).transpose(0, 2, 1)
    tgt_e = rows[2 * n_seq:2 * n_seq + _B].T
    ns_e = rows[2 * n_seq + _B:].reshape(_B, _NNS * _E).T

    preds, aux_loss = _tc_network(pos_tm, neg_tm, tgt_e, ns_e, seq_len, params)
    return (preds, aux_loss)
